# baseline (speedup 1.0000x reference)
import functools

import jax
import jax.numpy as jnp
from jax import lax
from jax.experimental import pallas as pl
from jax.experimental.pallas import tpu as pltpu
from jax.experimental.pallas import tpu_sc as plsc

_N = 10000
_E = 320000
_H = 128
_T = 4
_STEPS = [3, 1]

_NC = 2
_NS = 16
_C = 80
_R = 5
_GRP = _NS * _C * _R
_GMAX = -(-_E // _GRP)
_KCAP = _GMAX * _R * _C
_SPLIT = _N // 2
_NPH = 5120
_SCRAP = 5100
_RPT = _NPH // _NS


def _row_block(i):
    return (i, 0)


def _tc_table_body(s_ref, w_ref, b_ref, o_ref):
    o_ref[...] = (
        jnp.dot(s_ref[...], w_ref[...], preferred_element_type=jnp.float32)
        + b_ref[...]
    )


def _tc_table(states, wcat, bcat, blk):
    grid = (_N // blk,)
    return pl.pallas_call(
        _tc_table_body,
        grid=grid,
        in_specs=[
            pl.BlockSpec((blk, _H), _row_block),
            pl.BlockSpec((_H, _T * _H), lambda i: (0, 0)),
            pl.BlockSpec((1, _T * _H), lambda i: (0, 0)),
        ],
        out_specs=pl.BlockSpec((blk, _T * _H), _row_block),
        out_shape=jax.ShapeDtypeStruct((_N, _T * _H), jnp.float32),
    )(states, wcat, bcat)


def _tc_gru_body(p_ref, h_ref, k_ref, r_ref, b_ref, o_ref):
    x = p_ref[0]
    h = h_ref[...]
    mx = jnp.dot(x, k_ref[...], preferred_element_type=jnp.float32) + b_ref[0:1, :]
    mi = jnp.dot(h, r_ref[...], preferred_element_type=jnp.float32) + b_ref[1:2, :]
    z = jax.nn.sigmoid(mx[:, :_H] + mi[:, :_H])
    r = jax.nn.sigmoid(mx[:, _H:2 * _H] + mi[:, _H:2 * _H])
    hh = jnp.tanh(mx[:, 2 * _H:] + r * mi[:, 2 * _H:])
    o_ref[...] = z * h + (1.0 - z) * hh


def _tc_gru(parts, h, gk, grk, gb, blk):
    grid = (_N // blk,)
    half = _SPLIT // blk
    return pl.pallas_call(
        _tc_gru_body,
        grid=grid,
        in_specs=[
            pl.BlockSpec((1, blk, _H), lambda i: (i // half, i % half, 0)),
            pl.BlockSpec((blk, _H), _row_block),
            pl.BlockSpec((_H, 3 * _H), lambda i: (0, 0)),
            pl.BlockSpec((_H, 3 * _H), lambda i: (0, 0)),
            pl.BlockSpec((2, 3 * _H), lambda i: (0, 0)),
        ],
        out_specs=pl.BlockSpec((blk, _H), _row_block),
        out_shape=jax.ShapeDtypeStruct((_N, _H), jnp.float32),
    )(parts, h, gk, grk, gb)


def _sc_edge_body(yb_hbm, key_hbm, dst_hbm, grps_hbm, zeros_hbm, out_hbm,
                  key_v, dsts, rows, gsems, dsems, zsem, grps_vm, acc_sh):
    cid = lax.axis_index("c")
    sid = lax.axis_index("s")
    zbase = sid * _RPT
    pltpu.async_copy(zeros_hbm.at[pl.ds(zbase, _RPT)],
                     acc_sh.at[pl.ds(zbase, _RPT)], zsem)
    pltpu.sync_copy(grps_hbm, grps_vm)
    ngrp = grps_vm[cid, :][0]
    nchunk = ngrp * _R
    ebase = cid * _E + sid * (nchunk * _C)
    pltpu.sync_copy(key_hbm.at[pl.ds(ebase, _KCAP)], key_v)
    pltpu.make_async_copy(zeros_hbm.at[pl.ds(0, _RPT)],
                          acc_sh.at[pl.ds(0, _RPT)], zsem).wait()
    plsc.subcore_barrier()

    def issue(k, j):
        off = pl.multiple_of(k * _C, 8)
        pltpu.async_copy(dst_hbm.at[pl.ds(ebase + off, _C)], dsts[j], dsems[j])
        pltpu.async_copy(yb_hbm.at[key_v.at[pl.ds(off, _C)]], rows[j], gsems[j])

    for j in range(_R):
        issue(j, j)

    def grp(gi, carry):
        for j in range(_R):
            k = gi * _R + j
            pltpu.make_async_copy(dst_hbm.at[pl.ds(0, _C)], dsts[j], dsems[j]).wait()
            pltpu.make_async_copy(yb_hbm.at[pl.ds(0, _C)], rows[j], gsems[j]).wait()
            pltpu.sync_copy(rows[j], acc_sh.at[dsts[j]], add=True)

            @pl.when(k + _R < nchunk)
            def _():
                issue(k + _R, j)
        return carry

    lax.fori_loop(0, ngrp, grp, 0)
    plsc.subcore_barrier()
    obase = cid * _NPH + sid * _RPT
    pltpu.sync_copy(acc_sh.at[pl.ds(zbase, _RPT)], out_hbm.at[pl.ds(obase, _RPT)])


@functools.partial(
    pl.kernel,
    out_type=jax.ShapeDtypeStruct((_NC * _NPH, _H), jnp.float32),
    mesh=plsc.VectorSubcoreMesh(core_axis_name="c", subcore_axis_name="s"),
    scratch_types=[
        pltpu.VMEM((_KCAP,), jnp.int32),
        [pltpu.VMEM((_C,), jnp.int32)] * _R,
        [pltpu.VMEM((_C, _H), jnp.float32)] * _R,
        [pltpu.SemaphoreType.DMA] * _R,
        [pltpu.SemaphoreType.DMA] * _R,
        pltpu.SemaphoreType.DMA,
        pltpu.VMEM((_NC, 16), jnp.int32),
        pltpu.VMEM_SHARED((_NPH, _H), jnp.float32),
    ],
)
def _sc_edge(yb, key, dst, grps, zeros, out,
             key_v, dsts, rows, gsems, dsems, zsem, grps_vm, acc_sh):
    _sc_edge_body(yb, key, dst, grps, zeros, out,
                  key_v, dsts, rows, gsems, dsems, zsem, grps_vm, acc_sh)


def _partition_edges(etype, src, dst):
    key = src * _T + etype
    right = dst >= _SPLIT
    n1 = jnp.sum(right.astype(jnp.int32))
    n0 = _E - n1
    rank = jnp.cumsum(right.astype(jnp.int32))
    idx = jnp.arange(_E, dtype=jnp.int32)
    lrank = idx + 1 - rank
    pos = jnp.where(right, _E + rank - 1, lrank - 1)
    key2 = jnp.zeros((2 * _E,), jnp.int32).at[pos].set(key)
    dstl2 = jnp.full((2 * _E,), _SCRAP, jnp.int32).at[pos].set(
        jnp.where(right, dst - _SPLIT, dst))
    grps = jnp.maximum(
        -(-jnp.stack([n0, n1]) // _GRP), 1).astype(jnp.int32)
    grps16 = grps[:, None] * jnp.eye(1, 16, dtype=jnp.int32)
    return key2, dstl2, grps16


def kernel(states, edges, type_weights, type_biases, gru_kernel,
           gru_rec_kernel, gru_bias):
    etype = edges[:, 0].astype(jnp.int32)
    src = edges[:, 1].astype(jnp.int32)
    dst = edges[:, 2].astype(jnp.int32)
    key2, dstl2, grps = _partition_edges(etype, src, dst)
    zeros = jnp.zeros((_NPH, _H), jnp.float32)

    h = states
    for layer, steps in enumerate(_STEPS):
        wcat = jnp.transpose(type_weights[layer], (1, 0, 2)).reshape(_H, _T * _H)
        bcat = type_biases[layer].reshape(1, _T * _H)
        gk = gru_kernel[layer]
        grk = gru_rec_kernel[layer]
        gb = gru_bias[layer]
        for _ in range(steps):
            yb = _tc_table(h, wcat, bcat, 1000)
            yb_flat = yb.reshape(_N * _T, _H)
            parts = _sc_edge(yb_flat, key2, dstl2, grps, zeros)
            h = _tc_gru(parts.reshape(_NC, _NPH, _H), h, gk, grk, gb, 1000)
    return h


# speedup vs baseline: 6.5702x; 6.5702x over previous
import functools

import jax
import jax.numpy as jnp
from jax import lax
from jax.experimental import pallas as pl
from jax.experimental.pallas import tpu as pltpu
from jax.experimental.pallas import tpu_sc as plsc

_N = 10000
_E = 320000
_H = 128
_T = 4
_STEPS = [3, 1]

_NC = 2
_NS = 16
_NW = _NC * _NS
_EW = _E // _NW
_C = 40
_NCHUNK = _EW // _C
_R = 5
_NG = _NCHUNK // _R
_NP = 10240
_RPT = _NP // _NS


def _row_block(i):
    return (i, 0)


def _tc_pre_body(h_ref, w_ref, b_ref, r_ref, rb_ref, yb_ref, mi_ref):
    h = h_ref[...]
    yb_ref[...] = (
        jnp.dot(h, w_ref[...], preferred_element_type=jnp.float32) + b_ref[...]
    )
    mi_ref[...] = (
        jnp.dot(h, r_ref[...], preferred_element_type=jnp.float32) + rb_ref[...]
    )


def _tc_pre(h, wcat, bcat, grk, gb1, blk):
    grid = (_N // blk,)
    return pl.pallas_call(
        _tc_pre_body,
        grid=grid,
        in_specs=[
            pl.BlockSpec((blk, _H), _row_block),
            pl.BlockSpec((_H, _T * _H), lambda i: (0, 0)),
            pl.BlockSpec((1, _T * _H), lambda i: (0, 0)),
            pl.BlockSpec((_H, 3 * _H), lambda i: (0, 0)),
            pl.BlockSpec((1, 3 * _H), lambda i: (0, 0)),
        ],
        out_specs=[
            pl.BlockSpec((blk, _T * _H), _row_block),
            pl.BlockSpec((blk, 3 * _H), _row_block),
        ],
        out_shape=[
            jax.ShapeDtypeStruct((_N, _T * _H), jnp.float32),
            jax.ShapeDtypeStruct((_N, 3 * _H), jnp.float32),
        ],
    )(h, wcat, bcat, grk, gb1)


def _tc_gru_body(p_ref, mi_ref, h_ref, k_ref, b_ref, o_ref):
    x = p_ref[0] + p_ref[1]
    h = h_ref[...]
    mx = jnp.dot(x, k_ref[...], preferred_element_type=jnp.float32) + b_ref[...]
    mi = mi_ref[...]
    z = jax.nn.sigmoid(mx[:, :_H] + mi[:, :_H])
    r = jax.nn.sigmoid(mx[:, _H:2 * _H] + mi[:, _H:2 * _H])
    hh = jnp.tanh(mx[:, 2 * _H:] + r * mi[:, 2 * _H:])
    o_ref[...] = z * h + (1.0 - z) * hh


def _tc_gru(parts, mi, h, gk, gb0, blk):
    grid = (_N // blk,)
    return pl.pallas_call(
        _tc_gru_body,
        grid=grid,
        in_specs=[
            pl.BlockSpec((2, blk, _H), lambda i: (0, i, 0)),
            pl.BlockSpec((blk, 3 * _H), _row_block),
            pl.BlockSpec((blk, _H), _row_block),
            pl.BlockSpec((_H, 3 * _H), lambda i: (0, 0)),
            pl.BlockSpec((1, 3 * _H), lambda i: (0, 0)),
        ],
        out_specs=pl.BlockSpec((blk, _H), _row_block),
        out_shape=jax.ShapeDtypeStruct((_N, _H), jnp.float32),
    )(parts, mi, h, gk, gb0)


def _sc_edge_body(yb_hbm, key_hbm, dst_hbm, zeros_hbm, out_hbm,
                  key_v, dsts, rows, gsems, dsems, acc_sh):
    cid = lax.axis_index("c")
    sid = lax.axis_index("s")
    zbase = sid * _RPT
    pltpu.sync_copy(zeros_hbm.at[pl.ds(zbase, _RPT)], acc_sh.at[pl.ds(zbase, _RPT)])

    wid = cid * _NS + sid
    ebase = wid * _EW
    pltpu.sync_copy(key_hbm.at[pl.ds(ebase, _EW)], key_v)
    plsc.subcore_barrier()

    def issue(k, j):
        off = pl.multiple_of(k * _C, 8)
        pltpu.async_copy(dst_hbm.at[pl.ds(ebase + off, _C)], dsts[j], dsems[j])
        pltpu.async_copy(yb_hbm.at[key_v.at[pl.ds(off, _C)]], rows[j], gsems[j])

    for j in range(_R):
        issue(j, j)

    def grp(gi, carry):
        for j in range(_R):
            k = gi * _R + j
            koff = pl.multiple_of(k * _C, 8)
            pltpu.make_async_copy(dst_hbm.at[pl.ds(0, _C)], dsts[j], dsems[j]).wait()
            pltpu.make_async_copy(yb_hbm.at[key_v.at[pl.ds(koff, _C)]],
                                  rows[j], gsems[j]).wait()
            pltpu.sync_copy(rows[j], acc_sh.at[dsts[j]], add=True)

            @pl.when(k + _R < _NCHUNK)
            def _():
                issue(k + _R, j)
        return carry

    lax.fori_loop(0, _NG, grp, 0)
    plsc.subcore_barrier()
    obase = cid * _NP + sid * _RPT
    pltpu.sync_copy(acc_sh.at[pl.ds(zbase, _RPT)], out_hbm.at[pl.ds(obase, _RPT)])


@functools.partial(
    pl.kernel,
    out_type=jax.ShapeDtypeStruct((_NC * _NP, _H), jnp.float32),
    mesh=plsc.VectorSubcoreMesh(core_axis_name="c", subcore_axis_name="s"),
    scratch_types=[
        pltpu.VMEM((_EW,), jnp.int32),
        [pltpu.VMEM((_C,), jnp.int32)] * _R,
        [pltpu.VMEM((_C, _H), jnp.float32)] * _R,
        [pltpu.SemaphoreType.DMA] * _R,
        [pltpu.SemaphoreType.DMA] * _R,
        pltpu.VMEM_SHARED((_NP, _H), jnp.float32),
    ],
)
def _sc_edge(yb, key, dst, zeros, out, key_v, dsts, rows, gsems, dsems, acc_sh):
    _sc_edge_body(yb, key, dst, zeros, out, key_v, dsts, rows, gsems, dsems, acc_sh)


def kernel(states, edges, type_weights, type_biases, gru_kernel,
           gru_rec_kernel, gru_bias):
    etype = edges[:, 0].astype(jnp.int32)
    src = edges[:, 1].astype(jnp.int32)
    dst = edges[:, 2].astype(jnp.int32)
    key = src * _T + etype
    zeros = jnp.zeros((_NP, _H), jnp.float32)

    h = states
    for layer, steps in enumerate(_STEPS):
        wcat = jnp.transpose(type_weights[layer], (1, 0, 2)).reshape(_H, _T * _H)
        bcat = type_biases[layer].reshape(1, _T * _H)
        gk = gru_kernel[layer]
        grk = gru_rec_kernel[layer]
        gb0 = gru_bias[layer, 0].reshape(1, 3 * _H)
        gb1 = gru_bias[layer, 1].reshape(1, 3 * _H)
        for _ in range(steps):
            yb, mi = _tc_pre(h, wcat, bcat, grk, gb1, 1000)
            yb_flat = yb.reshape(_N * _T, _H)
            parts = _sc_edge(yb_flat, key, dst, zeros)
            h = _tc_gru(parts.reshape(_NC, _NP, _H), mi, h, gk, gb0, 1000)
    return h
